# Initial kernel scaffold; baseline (speedup 1.0000x reference)
import jax
import jax.numpy as jnp
from jax.experimental import pallas as pl


def kernel(fv, fe, fg, fv_pos, edge_index, params):
    raise NotImplementedError("write your pallas kernel here")


# speedup vs baseline: 6.7467x; 6.7467x over previous
import jax
import jax.numpy as jnp
from jax.experimental import pallas as pl

N = 10000; E = 320000; B = 16; NP = 625; EP = 20000
DV = 128; DE = 128; DG = 128; DGEO = 16; NH = 4; DH = 32
TE = 2000
T = EP // TE

_F32 = jnp.float32


def _gelu(x):
    return 0.5 * x * (1.0 + jax.lax.erf(x * 0.7071067811865476))


def _mm(a, b):
    return jax.lax.dot_general(a, b, (((1,), (0,)), ((), ())),
                               preferred_element_type=_F32)


def _mmT(a, b):
    return jax.lax.dot_general(a, b, (((0,), (0,)), ((), ())),
                               preferred_element_type=_F32)


def _onehot(idx):
    iota = jax.lax.broadcasted_iota(jnp.int32, (idx.shape[0], NP), 1)
    return (iota == idx[:, None]).astype(_F32)


def _k0(fg_ref, fv_ref, pos_ref,
        Wg2v_ref, bg2v_ref, Wvu_ref, bvu_ref,
        Wga0_ref, bga0_ref, Wga1_ref, Weug_ref,
        fvp_ref, vu_ref, G_ref, garow_ref, eug_ref):
    fg = fg_ref[0]
    y = _mm(fg, Wg2v_ref[...]) + bg2v_ref[...]
    gsc = y[:, :DV]
    gsh = y[:, DV:]
    fv = fv_ref[0]
    fvp = fv * (1.0 + gsc) + gsh
    fvp_ref[0] = fvp
    vu_ref[0] = _mm(fvp, Wvu_ref[...]) + bvu_ref[...]
    garow_ref[0] = _mm(_gelu(_mm(fg, Wga0_ref[...]) + bga0_ref[...]),
                       Wga1_ref[...])
    eug_ref[0] = _mm(fg, Weug_ref[...])
    pos = pos_ref[0]
    dist = jnp.sqrt(jnp.sum(pos * pos, axis=1, keepdims=True))
    G_ref[0] = jnp.concatenate([pos, dist], axis=1)


def _k1(fe_ref, sl_ref, dl_ref, fvp_ref, vu_ref, G_ref, garow_ref,
        Wg0_ref, bg0_ref, Wg1_ref, bg1_ref, Wg2_ref,
        Wea0_ref, bea0_ref, Wea1_ref, bea1_ref,
        Wm0a_ref, Wm0b_ref, bm0_ref, Wm1_ref, bm1_ref,
        agg_ref, geo_ref, w4_ref, gsum_ref, gmin_ref, gmax_ref):
    t = pl.program_id(1)
    sl = sl_ref[0, 0, :]
    dl = dl_ref[0, 0, :]
    oh_s = _onehot(sl)
    oh_d = _onehot(dl)
    G = G_ref[0]
    Gs = _mm(oh_s, G)
    Gd = _mm(oh_d, G)
    diff = Gs[:, 0:3] - Gd[:, 0:3]
    fe_dist = jnp.sqrt(jnp.sum(diff * diff, axis=1, keepdims=True))
    Wg0 = Wg0_ref[...]
    h = (fe_dist * Wg0[0:1, :] + Gs[:, 3:4] * Wg0[1:2, :]
         + Gd[:, 3:4] * Wg0[2:3, :] + bg0_ref[...])
    h = _gelu(h)
    h = _gelu(_mm(h, Wg1_ref[...]) + bg1_ref[...])
    fe_geo = _mm(h, Wg2_ref[...])
    geo_ref[0] = fe_geo
    w4_ref[0] = jnp.concatenate(
        [diff / (fe_dist + 1.0), jnp.zeros((TE, 1), _F32)], axis=1)
    eg = _gelu(_mm(fe_geo, Wea0_ref[...]) + bea0_ref[...])
    gate = _mm(eg, Wea1_ref[...]) + bea1_ref[...] + garow_ref[0]
    fe_t = fe_ref[0]
    fv_s = _mm(oh_s, fvp_ref[0])
    h = _gelu(_mm(fv_s, Wm0a_ref[...]) + _mm(fe_t, Wm0b_ref[...])
              + bm0_ref[...])
    msg = _gelu(_mm(h, Wm1_ref[...]) + bm1_ref[...])
    vud = _mm(oh_d, vu_ref[0])
    m = _gelu(vud[:, :DV] * msg + vud[:, DV:])
    mw = (m.reshape(TE, NH, DH) * gate[:, :, None]).reshape(TE, NH * DH)
    contrib = _mmT(oh_d, mw)
    tsum = jnp.sum(fe_geo, axis=0, keepdims=True)
    tmin = jnp.min(fe_geo, axis=0, keepdims=True)
    tmax = jnp.max(fe_geo, axis=0, keepdims=True)

    @pl.when(t == 0)
    def _():
        agg_ref[0] = contrib
        gsum_ref[0] = tsum
        gmin_ref[0] = tmin
        gmax_ref[0] = tmax

    @pl.when(t != 0)
    def _():
        agg_ref[0] += contrib
        gsum_ref[0] += tsum
        gmin_ref[0] = jnp.minimum(gmin_ref[0], tmin)
        gmax_ref[0] = jnp.maximum(gmax_ref[0], tmax)


def _k2(agg_ref, Wproj_ref, bproj_ref, Wa_ref, ba_ref, Wb_ref, bb_ref,
        fvn_ref, a_ref, b_ref, vsum_ref, vmin_ref, vmax_ref):
    fvn = _mm(agg_ref[0], Wproj_ref[...]) + bproj_ref[...]
    fvn_ref[0] = fvn
    a_ref[0] = _mm(fvn, Wa_ref[...]) + ba_ref[...]
    b_ref[0] = _mm(fvn, Wb_ref[...]) + bb_ref[...]
    vsum_ref[0] = jnp.sum(fvn, axis=0, keepdims=True)
    vmin_ref[0] = jnp.min(fvn, axis=0, keepdims=True)
    vmax_ref[0] = jnp.max(fvn, axis=0, keepdims=True)


def _k3(fe_ref, geo_ref, w4_ref, sl_ref, dl_ref, a_ref, b_ref, eug_ref,
        We0a_ref, We0g_ref, be0_ref, We1_ref, be1_ref, We2_ref, be2_ref,
        Wu0_ref, bu0_ref, Wu1_ref, bu1_ref,
        Wp0_ref, bp0_ref, Wp1_ref, bp1_ref, wp2_ref,
        fen_ref, pagg_ref, esum_ref, emin_ref, emax_ref):
    t = pl.program_id(1)
    sl = sl_ref[0, 0, :]
    dl = dl_ref[0, 0, :]
    oh_s = _onehot(sl)
    oh_d = _onehot(dl)
    ab = _mm(oh_s, a_ref[0]) * _mm(oh_d, b_ref[0])
    geo = geo_ref[0]
    h = _gelu(_mm(ab, We0a_ref[...]) + _mm(geo, We0g_ref[...])
              + eug_ref[0] + be0_ref[...])
    h = _gelu(_mm(h, We1_ref[...]) + be1_ref[...])
    eo = _mm(h, We2_ref[...]) + be2_ref[...]
    esh = eo[:, :DE]
    esc = eo[:, DE:]
    fe_t = fe_ref[0]
    u = _gelu(_mm(fe_t, Wu0_ref[...]) + bu0_ref[...])
    fen = (_mm(u, Wu1_ref[...]) + bu1_ref[...]) * (esc + 1.0) + esh
    fen_ref[0] = fen
    h = _gelu(_mm(fen, Wp0_ref[...]) + bp0_ref[...])
    h = _gelu(_mm(h, Wp1_ref[...]) + bp1_ref[...])
    pm = jnp.sum(h * wp2_ref[...], axis=1, keepdims=True)
    pc = pm * w4_ref[0]
    pcon = _mmT(oh_d, pc)
    tsum = jnp.sum(fen, axis=0, keepdims=True)
    tmin = jnp.min(fen, axis=0, keepdims=True)
    tmax = jnp.max(fen, axis=0, keepdims=True)

    @pl.when(t == 0)
    def _():
        pagg_ref[0] = pcon
        esum_ref[0] = tsum
        emin_ref[0] = tmin
        emax_ref[0] = tmax

    @pl.when(t != 0)
    def _():
        pagg_ref[0] += pcon
        esum_ref[0] += tsum
        emin_ref[0] = jnp.minimum(emin_ref[0], tmin)
        emax_ref[0] = jnp.maximum(emax_ref[0], tmax)


def _k4(fg_ref, vsum_ref, vmin_ref, vmax_ref, esum_ref, emin_ref, emax_ref,
        gsum_ref, gmin_ref, gmax_ref, pos_ref, pagg_ref,
        Wvr1_ref, bvr1_ref, Wvr2_ref, bvr2_ref, Wvr3_ref, bvr3_ref,
        Wer1_ref, ber1_ref, Wer2_ref, ber2_ref, Wer3_ref, ber3_ref,
        Wgr1_ref, bgr1_ref, Wgr2_ref, bgr2_ref, Wgr3_ref, bgr3_ref,
        Wq0a_ref, Wq0b_ref, Wq0c_ref, q0n_ref, bq0_ref, Wq1_ref, bq1_ref,
        Wih_ref, bih_ref, Whh_ref, bhh_ref,
        fgn_ref, posn_ref):
    fg = fg_ref[...].reshape(B, DG)
    vmean = vsum_ref[...].reshape(B, DV) * (1.0 / NP)
    vmin = vmin_ref[...].reshape(B, DV)
    vmax = vmax_ref[...].reshape(B, DV)
    emean = esum_ref[...].reshape(B, DE) * (1.0 / EP)
    emin = emin_ref[...].reshape(B, DE)
    emax = emax_ref[...].reshape(B, DE)
    gmean = gsum_ref[...].reshape(B, DGEO) * (1.0 / EP)
    gmin = gmin_ref[...].reshape(B, DGEO)
    gmax = gmax_ref[...].reshape(B, DGEO)
    fv2g = (_mm(vmean, Wvr1_ref[...]) + bvr1_ref[...]
            + _mm(vmin, Wvr2_ref[...]) + bvr2_ref[...]
            + _mm(vmax, Wvr3_ref[...]) + bvr3_ref[...])
    fe2g = (_mm(emean, Wer1_ref[...]) + ber1_ref[...]
            + _mm(emin, Wer2_ref[...]) + ber2_ref[...]
            + _mm(emax, Wer3_ref[...]) + ber3_ref[...])
    fgeo2g = (_mm(gmean, Wgr1_ref[...]) + bgr1_ref[...]
              + _mm(gmin, Wgr2_ref[...]) + bgr2_ref[...]
              + _mm(gmax, Wgr3_ref[...]) + bgr3_ref[...])
    hx = _gelu(_mm(fv2g, Wq0a_ref[...]) + _mm(fe2g, Wq0b_ref[...])
               + _mm(fgeo2g, Wq0c_ref[...]) + q0n_ref[...] + bq0_ref[...])
    x = _mm(hx, Wq1_ref[...]) + bq1_ref[...]
    gi = _mm(x, Wih_ref[...]) + bih_ref[...]
    gh = _mm(fg, Whh_ref[...]) + bhh_ref[...]
    r = jax.nn.sigmoid(gi[:, :DG] + gh[:, :DG])
    z = jax.nn.sigmoid(gi[:, DG:2 * DG] + gh[:, DG:2 * DG])
    n = jnp.tanh(gi[:, 2 * DG:] + r * gh[:, 2 * DG:])
    fgn_ref[...] = ((1.0 - z) * n + z * fg).reshape(B, 1, DG)
    posn_ref[...] = pos_ref[...] + pagg_ref[...][:, :, 0:3]


def kernel(fv, fe, fg, fv_pos, edge_index, params):
    p = params
    f32 = _F32

    def W(name):
        return p[name]['W'].T.astype(f32)

    def bias(name):
        return p[name]['b'].reshape(1, -1).astype(f32)

    fv3 = fv.reshape(B, NP, DV)
    pos3 = fv_pos.reshape(B, NP, 3)
    fg3 = fg.reshape(B, 1, DG)
    fe_r = fe.reshape(B * T, TE, DE)
    sl = (edge_index[0] % NP).astype(jnp.int32).reshape(B * T, 1, TE)
    dl = (edge_index[1] % NP).astype(jnp.int32).reshape(B * T, 1, TE)

    def bs(shape, imap):
        return pl.BlockSpec(shape, imap)

    g1 = lambda g: (g, 0, 0)
    w2 = lambda g: (0, 0)

    eu1_0W = p['eu1_0']['W']
    out0 = pl.pallas_call(
        _k0,
        grid=(B,),
        in_specs=[bs((1, 1, DG), g1), bs((1, NP, DV), g1), bs((1, NP, 3), g1),
                  bs((DG, 2 * DV), w2), bs((1, 2 * DV), w2),
                  bs((DV, 2 * DV), w2), bs((1, 2 * DV), w2),
                  bs((DG, DG), w2), bs((1, DG), w2), bs((DG, NH), w2),
                  bs((DG, DE), w2)],
        out_specs=[bs((1, NP, DV), g1), bs((1, NP, 2 * DV), g1),
                   bs((1, NP, 4), g1), bs((1, 1, NH), g1), bs((1, 1, DE), g1)],
        out_shape=[jax.ShapeDtypeStruct((B, NP, DV), f32),
                   jax.ShapeDtypeStruct((B, NP, 2 * DV), f32),
                   jax.ShapeDtypeStruct((B, NP, 4), f32),
                   jax.ShapeDtypeStruct((B, 1, NH), f32),
                   jax.ShapeDtypeStruct((B, 1, DE), f32)],
    )(fg3, fv3, pos3,
      W('lin_g2v'), bias('lin_g2v'), W('lin_v_upd'), bias('lin_v_upd'),
      W('ga0'), bias('ga0'), W('ga1'),
      eu1_0W[:, DE + DGEO:].T.astype(f32))
    fvp3, vu3, G3, garow, eug = out0

    g2 = lambda g, t: (g, 0, 0)
    e2 = lambda g, t: (g * T + t, 0, 0)
    w2b = lambda g, t: (0, 0)
    out1 = pl.pallas_call(
        _k1,
        grid=(B, T),
        in_specs=[bs((1, TE, DE), e2), bs((1, 1, TE), e2), bs((1, 1, TE), e2),
                  bs((1, NP, DV), g2), bs((1, NP, 2 * DV), g2),
                  bs((1, NP, 4), g2), bs((1, 1, NH), g2),
                  bs((3, DGEO), w2b), bs((1, DGEO), w2b),
                  bs((DGEO, DGEO), w2b), bs((1, DGEO), w2b),
                  bs((DGEO, DGEO), w2b),
                  bs((DGEO, DGEO), w2b), bs((1, DGEO), w2b),
                  bs((DGEO, NH), w2b), bs((1, NH), w2b),
                  bs((DV, DV), w2b), bs((DE, DV), w2b), bs((1, DV), w2b),
                  bs((DV, DV), w2b), bs((1, DV), w2b)],
        out_specs=[bs((1, NP, NH * DH), g2), bs((1, TE, DGEO), e2),
                   bs((1, TE, 4), e2), bs((1, 1, DGEO), g2),
                   bs((1, 1, DGEO), g2), bs((1, 1, DGEO), g2)],
        out_shape=[jax.ShapeDtypeStruct((B, NP, NH * DH), f32),
                   jax.ShapeDtypeStruct((B * T, TE, DGEO), f32),
                   jax.ShapeDtypeStruct((B * T, TE, 4), f32),
                   jax.ShapeDtypeStruct((B, 1, DGEO), f32),
                   jax.ShapeDtypeStruct((B, 1, DGEO), f32),
                   jax.ShapeDtypeStruct((B, 1, DGEO), f32)],
    )(fe_r, sl, dl, fvp3, vu3, G3, garow,
      W('geo0'), bias('geo0'), W('geo1'), bias('geo1'), W('geo2'),
      W('ega0'), bias('ega0'), W('ega1'), bias('ega1'),
      p['vmsg0']['W'][:, :DV].T.astype(f32),
      p['vmsg0']['W'][:, DV:].T.astype(f32), bias('vmsg0'),
      W('vmsg1'), bias('vmsg1'))
    agg3, geo_r, w4_r, gsum, gmin, gmax = out1

    out2 = pl.pallas_call(
        _k2,
        grid=(B,),
        in_specs=[bs((1, NP, NH * DH), g1),
                  bs((NH * DH, DV), w2), bs((1, DV), w2),
                  bs((DV, DE), w2), bs((1, DE), w2),
                  bs((DV, DE), w2), bs((1, DE), w2)],
        out_specs=[bs((1, NP, DV), g1), bs((1, NP, DE), g1),
                   bs((1, NP, DE), g1), bs((1, 1, DV), g1),
                   bs((1, 1, DV), g1), bs((1, 1, DV), g1)],
        out_shape=[jax.ShapeDtypeStruct((B, NP, DV), f32),
                   jax.ShapeDtypeStruct((B, NP, DE), f32),
                   jax.ShapeDtypeStruct((B, NP, DE), f32),
                   jax.ShapeDtypeStruct((B, 1, DV), f32),
                   jax.ShapeDtypeStruct((B, 1, DV), f32),
                   jax.ShapeDtypeStruct((B, 1, DV), f32)],
    )(agg3, W('v_upd_proj'), bias('v_upd_proj'),
      W('v2e1'), bias('v2e1'), W('v2e2'), bias('v2e2'))
    fvn3, a3, b3, vsum, vmin, vmax = out2

    out3 = pl.pallas_call(
        _k3,
        grid=(B, T),
        in_specs=[bs((1, TE, DE), e2), bs((1, TE, DGEO), e2),
                  bs((1, TE, 4), e2), bs((1, 1, TE), e2), bs((1, 1, TE), e2),
                  bs((1, NP, DE), g2), bs((1, NP, DE), g2), bs((1, 1, DE), g2),
                  bs((DE, DE), w2b), bs((DGEO, DE), w2b), bs((1, DE), w2b),
                  bs((DE, DE), w2b), bs((1, DE), w2b),
                  bs((DE, 2 * DE), w2b), bs((1, 2 * DE), w2b),
                  bs((DE, DE), w2b), bs((1, DE), w2b),
                  bs((DE, DE), w2b), bs((1, DE), w2b),
                  bs((DE, DE), w2b), bs((1, DE), w2b),
                  bs((DE, DE), w2b), bs((1, DE), w2b), bs((1, DE), w2b)],
        out_specs=[bs((1, TE, DE), e2), bs((1, NP, 4), g2),
                   bs((1, 1, DE), g2), bs((1, 1, DE), g2), bs((1, 1, DE), g2)],
        out_shape=[jax.ShapeDtypeStruct((B * T, TE, DE), f32),
                   jax.ShapeDtypeStruct((B, NP, 4), f32),
                   jax.ShapeDtypeStruct((B, 1, DE), f32),
                   jax.ShapeDtypeStruct((B, 1, DE), f32),
                   jax.ShapeDtypeStruct((B, 1, DE), f32)],
    )(fe_r, geo_r, w4_r, sl, dl, a3, b3, eug,
      eu1_0W[:, :DE].T.astype(f32), eu1_0W[:, DE:DE + DGEO].T.astype(f32),
      bias('eu1_0'), W('eu1_1'), bias('eu1_1'), W('eu1_2'), bias('eu1_2'),
      W('eu2_0'), bias('eu2_0'), W('eu2_1'), bias('eu2_1'),
      W('pm0'), bias('pm0'), W('pm1'), bias('pm1'),
      p['pm2']['W'].astype(f32))
    fen_r, pagg3, esum, emin, emax = out3

    q0W = p['gmlp0']['W']
    HG = DG // 2
    out4 = pl.pallas_call(
        _k4,
        out_shape=[jax.ShapeDtypeStruct((B, 1, DG), f32),
                   jax.ShapeDtypeStruct((B, NP, 3), f32)],
    )(fg3, vsum, vmin, vmax, esum, emin, emax, gsum, gmin, gmax,
      pos3, pagg3,
      W('vr1'), bias('vr1'), W('vr2'), bias('vr2'), W('vr3'), bias('vr3'),
      W('er1'), bias('er1'), W('er2'), bias('er2'), W('er3'), bias('er3'),
      W('gr1'), bias('gr1'), W('gr2'), bias('gr2'), W('gr3'), bias('gr3'),
      q0W[:, :DG].T.astype(f32), q0W[:, DG:DG + HG].T.astype(f32),
      q0W[:, DG + HG:2 * DG].T.astype(f32),
      (q0W[:, 2 * DG] * float(NP)).reshape(1, DG).astype(f32),
      bias('gmlp0'), W('gmlp1'), bias('gmlp1'),
      p['gru_Wih'].T.astype(f32), p['gru_bih'].reshape(1, -1).astype(f32),
      p['gru_Whh'].T.astype(f32), p['gru_bhh'].reshape(1, -1).astype(f32))
    fgn3, posn3 = out4

    return (fvn3.reshape(N, DV), fen_r.reshape(E, DE),
            fgn3.reshape(B, DG), posn3.reshape(N, 3))
